# baseline (speedup 1.0000x reference)
import functools

import jax
import jax.numpy as jnp
from jax import lax
from jax.experimental import pallas as pl
from jax.experimental.pallas import tpu as pltpu
from jax.experimental.pallas import tpu_sc as plsc

_BUFFER_SIZE = 100000
_D = 128
_NUM_CHAINS = 4096
_N_NEW = 204
_N_OLD = _NUM_CHAINS - _N_NEW

_U32 = jnp.uint32


def _u32c(x):
    return _U32(x)


def _rotl(x, r):
    return (x << _u32c(r)) | (x >> _u32c(32 - r))


_ROT0 = (13, 15, 26, 6)
_ROT1 = (17, 29, 16, 24)


def _threefry(k0, k1, x0, x1):
    ks2 = k0 ^ k1 ^ _u32c(0x1BD11BDA)
    ks = (k0, k1, ks2)
    x0 = x0 + k0
    x1 = x1 + k1
    for g in range(5):
        for r in (_ROT0 if g % 2 == 0 else _ROT1):
            x0 = x0 + x1
            x1 = _rotl(x1, r)
            x1 = x1 ^ x0
        x0 = x0 + ks[(g + 1) % 3]
        x1 = x1 + ks[(g + 2) % 3] + _u32c(g + 1)
    return x0, x1


def _uniform_chunk(s10, s11, zeros, lanes_u, flat_base):
    i = flat_base.astype(_U32) + lanes_u
    o0, o1 = _threefry(s10, s11, zeros, i)
    b = ((o0 ^ o1) >> _u32c(9)) | _u32c(0x3F800000)
    f = lax.bitcast_convert_type(b, jnp.float32) - jnp.float32(1.0)
    return jnp.maximum(jnp.float32(-1.0),
                       f * jnp.float32(2.0) + jnp.float32(-1.0))


def _sc_body(buf_hbm, key_hbm, out_hbm, kidx_v, kv, idx_v, new_v, mix_v,
             rows_v, sem, sem_c):
    wid = lax.axis_index("s") * 2 + lax.axis_index("c")
    zeros = jnp.zeros((16,), _U32)
    lanes = lax.broadcasted_iota(jnp.int32, (16,), 0)
    lanes_u = lanes.astype(_U32)

    kidx_v[pl.ds(0, 16)] = jnp.zeros((16,), jnp.int32)
    kidx_v[pl.ds(16, 16)] = jnp.ones((16,), jnp.int32)
    pltpu.async_copy(key_hbm.at[kidx_v], kv, sem).wait()
    k0s = kv[pl.ds(0, 16)].astype(_U32)
    k1s = kv[pl.ds(16, 16)].astype(_U32)

    ones = jnp.full((16,), _u32c(1))
    a0, a1 = _threefry(k0s, k1s, zeros, zeros)
    s10, s11 = _threefry(k0s, k1s, zeros, ones)
    s20, s21 = _threefry(a0, a1, zeros, ones)
    c20, c21 = _threefry(s20, s21, zeros, ones)

    span = jnp.full((16,), _u32c(_BUFFER_SIZE))

    old_base = jnp.where(wid == 31, 0, 4 + jnp.minimum(wid * 128, 3760))

    def _idx_chunk(i, carry):
        for k in range(4):
            c = i * 4 + k
            j = (old_base + c * 16).astype(_U32) + lanes_u
            o0, o1 = _threefry(c20, c21, zeros, j)
            idx_v[pl.ds(c * 16, 16)] = ((o0 ^ o1) % span).astype(jnp.int32)
        return carry

    lax.fori_loop(0, 2, _idx_chunk, 0)

    @pl.when(wid < 31)
    def _gather_start():
        pltpu.async_copy(buf_hbm.at[idx_v], rows_v, sem)

    @pl.when(wid == 31)
    def _gather_mixed_start():
        pltpu.async_copy(buf_hbm.at[idx_v.at[pl.ds(0, 4)]],
                         mix_v.at[pl.ds(4, 4)], sem)

    @pl.when(wid < 31)
    def _old_slab():
        start = 208 + jnp.minimum(wid * 128, 3760)
        pltpu.make_async_copy(buf_hbm.at[idx_v], rows_v, sem).wait()
        pltpu.async_copy(rows_v,
                         out_hbm.at[pl.ds(pl.multiple_of(start, 8), 128)],
                         sem_c)

    @pl.when(wid < 25)
    def _new_slab():
        start = wid * 8

        def _chunk(i, carry):
            for k in range(4):
                c = i * 4 + k
                new_v[c // 8, pl.ds((c % 8) * 16, 16)] = _uniform_chunk(
                    s10, s11, zeros, lanes_u, start * _D + c * 16)
            return carry

        lax.fori_loop(0, 16, _chunk, 0)
        pltpu.sync_copy(new_v,
                        out_hbm.at[pl.ds(pl.multiple_of(start, 8), 8)])

    @pl.when(wid == 31)
    def _mixed_block():
        def _chunk(i, carry):
            for k in range(4):
                c = i * 4 + k
                mix_v[c // 8, pl.ds((c % 8) * 16, 16)] = _uniform_chunk(
                    s10, s11, zeros, lanes_u, 200 * _D + c * 16)
            return carry

        lax.fori_loop(0, 8, _chunk, 0)
        pltpu.make_async_copy(buf_hbm.at[idx_v.at[pl.ds(0, 4)]],
                              mix_v.at[pl.ds(4, 4)], sem).wait()
        pltpu.sync_copy(mix_v, out_hbm.at[pl.ds(200, 8)])

    @pl.when(wid < 31)
    def _old_slab_drain():
        start = 208 + jnp.minimum(wid * 128, 3760)
        pltpu.make_async_copy(
            rows_v, out_hbm.at[pl.ds(pl.multiple_of(start, 8), 128)],
            sem_c).wait()


_mesh = plsc.VectorSubcoreMesh(core_axis_name="c", subcore_axis_name="s")

_sample = functools.partial(
    pl.kernel,
    mesh=_mesh,
    out_type=jax.ShapeDtypeStruct((_NUM_CHAINS, _D), jnp.float32),
    scratch_types=[
        pltpu.VMEM((32,), jnp.int32),
        pltpu.VMEM((32,), jnp.int32),
        pltpu.VMEM((128,), jnp.int32),
        pltpu.VMEM((8, _D), jnp.float32),
        pltpu.VMEM((8, _D), jnp.float32),
        pltpu.VMEM((128, _D), jnp.float32),
        pltpu.SemaphoreType.DMA,
        pltpu.SemaphoreType.DMA,
    ],
)(_sc_body)


def kernel(buffer, key):
    kd = lax.bitcast_convert_type(jax.random.key_data(key), jnp.int32)
    return _sample(buffer, kd)


# speedup vs baseline: 1.0365x; 1.0365x over previous
import functools

import jax
import jax.numpy as jnp
from jax import lax
from jax.experimental import pallas as pl
from jax.experimental.pallas import tpu as pltpu
from jax.experimental.pallas import tpu_sc as plsc

_BUFFER_SIZE = 100000
_D = 128
_NUM_CHAINS = 4096
_N_NEW = 204
_N_OLD = _NUM_CHAINS - _N_NEW

_U32 = jnp.uint32


def _u32c(x):
    return _U32(x)


def _rotl(x, r):
    return (x << _u32c(r)) | (x >> _u32c(32 - r))


_ROT0 = (13, 15, 26, 6)
_ROT1 = (17, 29, 16, 24)


def _threefry(k0, k1, x0, x1):
    ks2 = k0 ^ k1 ^ _u32c(0x1BD11BDA)
    ks = (k0, k1, ks2)
    x0 = x0 + k0
    x1 = x1 + k1
    for g in range(5):
        for r in (_ROT0 if g % 2 == 0 else _ROT1):
            x0 = x0 + x1
            x1 = _rotl(x1, r)
            x1 = x1 ^ x0
        x0 = x0 + ks[(g + 1) % 3]
        x1 = x1 + ks[(g + 2) % 3] + _u32c(g + 1)
    return x0, x1


def _uniform_chunk(s10, s11, zeros, lanes_u, flat_base):
    i = flat_base.astype(_U32) + lanes_u
    o0, o1 = _threefry(s10, s11, zeros, i)
    b = ((o0 ^ o1) >> _u32c(9)) | _u32c(0x3F800000)
    f = lax.bitcast_convert_type(b, jnp.float32) - jnp.float32(1.0)
    return jnp.maximum(jnp.float32(-1.0),
                       f * jnp.float32(2.0) + jnp.float32(-1.0))


def _sc_body(buf_hbm, key_hbm, out_hbm, kidx_v, kv, idx_v, new_v, mix_v,
             rows_v, sem, sem_c):
    wid = lax.axis_index("s") * 2 + lax.axis_index("c")
    zeros = jnp.zeros((16,), _U32)
    lanes = lax.broadcasted_iota(jnp.int32, (16,), 0)
    lanes_u = lanes.astype(_U32)

    kidx_v[pl.ds(0, 16)] = jnp.zeros((16,), jnp.int32)
    kidx_v[pl.ds(16, 16)] = jnp.ones((16,), jnp.int32)
    pltpu.async_copy(key_hbm.at[kidx_v], kv, sem).wait()
    k0s = kv[pl.ds(0, 16)].astype(_U32)
    k1s = kv[pl.ds(16, 16)].astype(_U32)

    ones = jnp.full((16,), _u32c(1))
    a0, a1 = _threefry(k0s, k1s, zeros, zeros)
    s10, s11 = _threefry(k0s, k1s, zeros, ones)
    s20, s21 = _threefry(a0, a1, zeros, ones)
    c20, c21 = _threefry(s20, s21, zeros, ones)

    span = jnp.full((16,), _u32c(_BUFFER_SIZE))

    old_base = jnp.where(wid == 31, 0, 4 + jnp.minimum(wid * 128, 3760))

    def _idx_chunk(i, carry):
        for k in range(4):
            c = i * 4 + k
            j = (old_base + c * 16).astype(_U32) + lanes_u
            o0, o1 = _threefry(c20, c21, zeros, j)
            idx_v[pl.ds(c * 16, 16)] = ((o0 ^ o1) % span).astype(jnp.int32)
        return carry

    lax.fori_loop(0, 2, _idx_chunk, 0)

    @pl.when(wid < 31)
    def _gather_start():
        pltpu.async_copy(buf_hbm.at[idx_v], rows_v, sem)

    @pl.when(wid == 31)
    def _gather_mixed_start():
        pltpu.async_copy(buf_hbm.at[idx_v.at[pl.ds(0, 4)]],
                         mix_v.at[pl.ds(4, 4)], sem)

    @pl.when(wid < 25)
    def _new_slab():
        start = wid * 8

        def _chunk(i, carry):
            for k in range(4):
                c = i * 4 + k
                new_v[c // 8, pl.ds((c % 8) * 16, 16)] = _uniform_chunk(
                    s10, s11, zeros, lanes_u, start * _D + c * 16)
            return carry

        lax.fori_loop(0, 16, _chunk, 0)
        pltpu.sync_copy(new_v,
                        out_hbm.at[pl.ds(pl.multiple_of(start, 8), 8)])

    @pl.when(wid == 31)
    def _mixed_block():
        def _chunk(i, carry):
            for k in range(4):
                c = i * 4 + k
                mix_v[c // 8, pl.ds((c % 8) * 16, 16)] = _uniform_chunk(
                    s10, s11, zeros, lanes_u, 200 * _D + c * 16)
            return carry

        lax.fori_loop(0, 8, _chunk, 0)
        pltpu.make_async_copy(buf_hbm.at[idx_v.at[pl.ds(0, 4)]],
                              mix_v.at[pl.ds(4, 4)], sem).wait()
        pltpu.sync_copy(mix_v, out_hbm.at[pl.ds(200, 8)])

    @pl.when(wid < 31)
    def _old_slab():
        start = 208 + jnp.minimum(wid * 128, 3760)
        pltpu.make_async_copy(buf_hbm.at[idx_v], rows_v, sem).wait()
        pltpu.sync_copy(rows_v,
                        out_hbm.at[pl.ds(pl.multiple_of(start, 8), 128)])


_mesh = plsc.VectorSubcoreMesh(core_axis_name="c", subcore_axis_name="s")

_sample = functools.partial(
    pl.kernel,
    mesh=_mesh,
    out_type=jax.ShapeDtypeStruct((_NUM_CHAINS, _D), jnp.float32),
    scratch_types=[
        pltpu.VMEM((32,), jnp.int32),
        pltpu.VMEM((32,), jnp.int32),
        pltpu.VMEM((128,), jnp.int32),
        pltpu.VMEM((8, _D), jnp.float32),
        pltpu.VMEM((8, _D), jnp.float32),
        pltpu.VMEM((128, _D), jnp.float32),
        pltpu.SemaphoreType.DMA,
        pltpu.SemaphoreType.DMA,
    ],
)(_sc_body)


def kernel(buffer, key):
    kd = lax.bitcast_convert_type(jax.random.key_data(key), jnp.int32)
    return _sample(buffer, kd)


# speedup vs baseline: 1.0813x; 1.0433x over previous
import functools

import jax
import jax.numpy as jnp
from jax import lax
from jax.experimental import pallas as pl
from jax.experimental.pallas import tpu as pltpu
from jax.experimental.pallas import tpu_sc as plsc

_BUFFER_SIZE = 100000
_D = 128
_NUM_CHAINS = 4096
_N_NEW = 204
_N_OLD = _NUM_CHAINS - _N_NEW

_U32 = jnp.uint32


def _u32c(x):
    return _U32(x)


def _rotl(x, r):
    return (x << _u32c(r)) | (x >> _u32c(32 - r))


_ROT0 = (13, 15, 26, 6)
_ROT1 = (17, 29, 16, 24)


def _threefry(k0, k1, x0, x1):
    ks2 = k0 ^ k1 ^ _u32c(0x1BD11BDA)
    ks = (k0, k1, ks2)
    x0 = x0 + k0
    x1 = x1 + k1
    for g in range(5):
        for r in (_ROT0 if g % 2 == 0 else _ROT1):
            x0 = x0 + x1
            x1 = _rotl(x1, r)
            x1 = x1 ^ x0
        x0 = x0 + ks[(g + 1) % 3]
        x1 = x1 + ks[(g + 2) % 3] + _u32c(g + 1)
    return x0, x1


def _uniform_chunk(s10, s11, zeros, lanes_u, flat_base):
    i = flat_base.astype(_U32) + lanes_u
    o0, o1 = _threefry(s10, s11, zeros, i)
    b = ((o0 ^ o1) >> _u32c(9)) | _u32c(0x3F800000)
    f = lax.bitcast_convert_type(b, jnp.float32) - jnp.float32(1.0)
    return jnp.maximum(jnp.float32(-1.0),
                       f * jnp.float32(2.0) + jnp.float32(-1.0))


def _sc_body(buf_hbm, key_hbm, out_hbm, kidx_v, kv, idx_v, new_v,
             rows_v, sem, sem_c):
    wid = lax.axis_index("s") * 2 + lax.axis_index("c")
    zeros = jnp.zeros((16,), _U32)
    lanes = lax.broadcasted_iota(jnp.int32, (16,), 0)
    lanes_u = lanes.astype(_U32)

    kidx_v[pl.ds(0, 16)] = jnp.zeros((16,), jnp.int32)
    kidx_v[pl.ds(16, 16)] = jnp.ones((16,), jnp.int32)
    pltpu.async_copy(key_hbm.at[kidx_v], kv, sem).wait()
    k0s = kv[pl.ds(0, 16)].astype(_U32)
    k1s = kv[pl.ds(16, 16)].astype(_U32)

    ones = jnp.full((16,), _u32c(1))
    a0, a1 = _threefry(k0s, k1s, zeros, zeros)
    s10, s11 = _threefry(k0s, k1s, zeros, ones)
    s20, s21 = _threefry(a0, a1, zeros, ones)
    c20, c21 = _threefry(s20, s21, zeros, ones)

    span = jnp.full((16,), _u32c(_BUFFER_SIZE))

    old_base = jnp.where(wid == 31, 0, 4 + jnp.minimum(wid * 128, 3760))

    def _idx_chunk(c, carry):
        j = (old_base + c * 16).astype(_U32) + lanes_u
        o0, o1 = _threefry(c20, c21, zeros, j)
        idx_v[pl.ds(c * 16, 16)] = ((o0 ^ o1) % span).astype(jnp.int32)
        return carry

    lax.fori_loop(0, 8, _idx_chunk, 0)

    @pl.when(wid < 31)
    def _gather_start():
        pltpu.async_copy(buf_hbm.at[idx_v], rows_v, sem)

    @pl.when(wid == 31)
    def _gather_mixed_start():
        pltpu.async_copy(buf_hbm.at[idx_v.at[pl.ds(0, 4)]],
                         new_v.at[pl.ds(4, 4)], sem)

    @pl.when((wid < 25) | (wid == 31))
    def _new_slab():
        start = jnp.where(wid == 31, 200, wid * 8)
        nchunks = jnp.where(wid == 31, 32, 64)

        def _chunk(c, carry):
            new_v[c // 8, pl.ds((c % 8) * 16, 16)] = _uniform_chunk(
                s10, s11, zeros, lanes_u, start * _D + c * 16)
            return carry

        lax.fori_loop(0, nchunks, _chunk, 0)

        @pl.when(wid == 31)
        def _mixed_wait():
            pltpu.make_async_copy(buf_hbm.at[idx_v.at[pl.ds(0, 4)]],
                                  new_v.at[pl.ds(4, 4)], sem).wait()

        pltpu.sync_copy(new_v,
                        out_hbm.at[pl.ds(pl.multiple_of(start, 8), 8)])

    @pl.when(wid < 31)
    def _old_slab():
        start = 208 + jnp.minimum(wid * 128, 3760)
        pltpu.make_async_copy(buf_hbm.at[idx_v], rows_v, sem).wait()
        pltpu.sync_copy(rows_v,
                        out_hbm.at[pl.ds(pl.multiple_of(start, 8), 128)])


_mesh = plsc.VectorSubcoreMesh(core_axis_name="c", subcore_axis_name="s")

_sample = functools.partial(
    pl.kernel,
    mesh=_mesh,
    out_type=jax.ShapeDtypeStruct((_NUM_CHAINS, _D), jnp.float32),
    scratch_types=[
        pltpu.VMEM((32,), jnp.int32),
        pltpu.VMEM((32,), jnp.int32),
        pltpu.VMEM((128,), jnp.int32),
        pltpu.VMEM((8, _D), jnp.float32),
        pltpu.VMEM((128, _D), jnp.float32),
        pltpu.SemaphoreType.DMA,
        pltpu.SemaphoreType.DMA,
    ],
)(_sc_body)


def kernel(buffer, key):
    kd = lax.bitcast_convert_type(jax.random.key_data(key), jnp.int32)
    return _sample(buffer, kd)


# speedup vs baseline: 1.2872x; 1.1904x over previous
import functools

import jax
import jax.numpy as jnp
from jax import lax
from jax.experimental import pallas as pl
from jax.experimental.pallas import tpu as pltpu
from jax.experimental.pallas import tpu_sc as plsc

_BUFFER_SIZE = 100000
_D = 128
_NUM_CHAINS = 4096
_N_NEW = 204
_N_OLD = _NUM_CHAINS - _N_NEW

_U32 = jnp.uint32


def _u32c(x):
    return _U32(x)


def _rotl(x, r):
    return (x << _u32c(r)) | (x >> _u32c(32 - r))


_ROT0 = (13, 15, 26, 6)
_ROT1 = (17, 29, 16, 24)


def _threefry(k0, k1, x0, x1):
    ks2 = k0 ^ k1 ^ _u32c(0x1BD11BDA)
    ks = (k0, k1, ks2)
    x0 = x0 + k0
    x1 = x1 + k1
    for g in range(5):
        for r in (_ROT0 if g % 2 == 0 else _ROT1):
            x0 = x0 + x1
            x1 = _rotl(x1, r)
            x1 = x1 ^ x0
        x0 = x0 + ks[(g + 1) % 3]
        x1 = x1 + ks[(g + 2) % 3] + _u32c(g + 1)
    return x0, x1


def _uniform_chunk(s10, s11, zeros, lanes_u, flat_base):
    i = flat_base.astype(_U32) + lanes_u
    o0, o1 = _threefry(s10, s11, zeros, i)
    b = ((o0 ^ o1) >> _u32c(9)) | _u32c(0x3F800000)
    f = lax.bitcast_convert_type(b, jnp.float32) - jnp.float32(1.0)
    return jnp.maximum(jnp.float32(-1.0),
                       f * jnp.float32(2.0) + jnp.float32(-1.0))


def _sc_body(buf_hbm, key_hbm, out_hbm, kidx_v, kv, idx_v, new_v,
             rows_v, sem, sem_c):
    wid = lax.axis_index("s") * 2 + lax.axis_index("c")
    zeros = jnp.zeros((16,), _U32)
    lanes = lax.broadcasted_iota(jnp.int32, (16,), 0)
    lanes_u = lanes.astype(_U32)

    pltpu.sync_copy(key_hbm, kv.at[pl.ds(0, 2)])
    kvec = kv[pl.ds(0, 16)]
    _dn = lax.GatherDimensionNumbers(
        offset_dims=(), collapsed_slice_dims=(0,), start_index_map=(0,))

    def _splat(lane):
        idx = jnp.full((16, 1), lane, jnp.int32)
        return lax.gather(kvec, idx, _dn, slice_sizes=(1,),
                          mode=lax.GatherScatterMode.PROMISE_IN_BOUNDS
                          ).astype(_U32)

    k0s = _splat(0)
    k1s = _splat(1)

    ones = jnp.full((16,), _u32c(1))
    a0, a1 = _threefry(k0s, k1s, zeros, zeros)
    s10, s11 = _threefry(k0s, k1s, zeros, ones)
    s20, s21 = _threefry(a0, a1, zeros, ones)
    c20, c21 = _threefry(s20, s21, zeros, ones)

    span = jnp.full((16,), _u32c(_BUFFER_SIZE))

    old_base = jnp.where(wid == 31, 0, 4 + jnp.minimum(wid * 128, 3760))

    def _idx_chunk(c, carry):
        j = (old_base + c * 16).astype(_U32) + lanes_u
        o0, o1 = _threefry(c20, c21, zeros, j)
        idx_v[pl.ds(c * 16, 16)] = ((o0 ^ o1) % span).astype(jnp.int32)
        return carry

    lax.fori_loop(0, 8, _idx_chunk, 0)

    @pl.when(wid < 31)
    def _gather_start():
        pltpu.async_copy(buf_hbm.at[idx_v], rows_v, sem)

    @pl.when(wid == 31)
    def _gather_mixed_start():
        pltpu.async_copy(buf_hbm.at[idx_v.at[pl.ds(0, 4)]],
                         new_v.at[pl.ds(4, 4)], sem)

    @pl.when((wid < 25) | (wid == 31))
    def _new_slab():
        start = jnp.where(wid == 31, 200, wid * 8)
        nchunks = jnp.where(wid == 31, 32, 64)

        def _chunk(c, carry):
            new_v[c // 8, pl.ds((c % 8) * 16, 16)] = _uniform_chunk(
                s10, s11, zeros, lanes_u, start * _D + c * 16)
            return carry

        lax.fori_loop(0, nchunks, _chunk, 0)

        @pl.when(wid == 31)
        def _mixed_wait():
            pltpu.make_async_copy(buf_hbm.at[idx_v.at[pl.ds(0, 4)]],
                                  new_v.at[pl.ds(4, 4)], sem).wait()

        pltpu.sync_copy(new_v,
                        out_hbm.at[pl.ds(pl.multiple_of(start, 8), 8)])

    @pl.when(wid < 31)
    def _old_slab():
        start = 208 + jnp.minimum(wid * 128, 3760)
        pltpu.make_async_copy(buf_hbm.at[idx_v], rows_v, sem).wait()
        pltpu.sync_copy(rows_v,
                        out_hbm.at[pl.ds(pl.multiple_of(start, 8), 128)])


_mesh = plsc.VectorSubcoreMesh(core_axis_name="c", subcore_axis_name="s")

_sample = functools.partial(
    pl.kernel,
    mesh=_mesh,
    out_type=jax.ShapeDtypeStruct((_NUM_CHAINS, _D), jnp.float32),
    scratch_types=[
        pltpu.VMEM((32,), jnp.int32),
        pltpu.VMEM((32,), jnp.int32),
        pltpu.VMEM((128,), jnp.int32),
        pltpu.VMEM((8, _D), jnp.float32),
        pltpu.VMEM((128, _D), jnp.float32),
        pltpu.SemaphoreType.DMA,
        pltpu.SemaphoreType.DMA,
    ],
)(_sc_body)


def kernel(buffer, key):
    kd = lax.bitcast_convert_type(jax.random.key_data(key), jnp.int32)
    return _sample(buffer, kd)


# speedup vs baseline: 1.2966x; 1.0073x over previous
import functools

import jax
import jax.numpy as jnp
from jax import lax
from jax.experimental import pallas as pl
from jax.experimental.pallas import tpu as pltpu
from jax.experimental.pallas import tpu_sc as plsc

_BUFFER_SIZE = 100000
_D = 128
_NUM_CHAINS = 4096
_N_NEW = 204
_N_OLD = _NUM_CHAINS - _N_NEW

_U32 = jnp.uint32


def _u32c(x):
    return _U32(x)


def _rotl(x, r):
    return (x << _u32c(r)) | (x >> _u32c(32 - r))


_ROT0 = (13, 15, 26, 6)
_ROT1 = (17, 29, 16, 24)


def _threefry(k0, k1, x0, x1):
    ks2 = k0 ^ k1 ^ _u32c(0x1BD11BDA)
    ks = (k0, k1, ks2)
    x0 = x0 + k0
    x1 = x1 + k1
    for g in range(5):
        for r in (_ROT0 if g % 2 == 0 else _ROT1):
            x0 = x0 + x1
            x1 = _rotl(x1, r)
            x1 = x1 ^ x0
        x0 = x0 + ks[(g + 1) % 3]
        x1 = x1 + ks[(g + 2) % 3] + _u32c(g + 1)
    return x0, x1


def _uniform_chunk(s10, s11, zeros, lanes_u, flat_base):
    i = flat_base.astype(_U32) + lanes_u
    o0, o1 = _threefry(s10, s11, zeros, i)
    b = ((o0 ^ o1) >> _u32c(9)) | _u32c(0x3F800000)
    f = lax.bitcast_convert_type(b, jnp.float32) - jnp.float32(1.0)
    return jnp.maximum(jnp.float32(-1.0),
                       f * jnp.float32(2.0) + jnp.float32(-1.0))


def _sc_body(buf_hbm, key_hbm, out_hbm, kv, idx_v, new_v,
             rows_v, sem, sem_c):
    wid = lax.axis_index("s") * 2 + lax.axis_index("c")
    zeros = jnp.zeros((16,), _U32)
    lanes = lax.broadcasted_iota(jnp.int32, (16,), 0)
    lanes_u = lanes.astype(_U32)

    pltpu.sync_copy(key_hbm, kv.at[pl.ds(0, 2)])
    kvec = kv[pl.ds(0, 16)]
    _dn = lax.GatherDimensionNumbers(
        offset_dims=(), collapsed_slice_dims=(0,), start_index_map=(0,))

    def _splat(lane):
        idx = jnp.full((16, 1), lane, jnp.int32)
        return lax.gather(kvec, idx, _dn, slice_sizes=(1,),
                          mode=lax.GatherScatterMode.PROMISE_IN_BOUNDS
                          ).astype(_U32)

    k0s = _splat(0)
    k1s = _splat(1)

    ones = jnp.full((16,), _u32c(1))
    a0, a1 = _threefry(k0s, k1s, zeros, zeros)
    s10, s11 = _threefry(k0s, k1s, zeros, ones)
    s20, s21 = _threefry(a0, a1, zeros, ones)
    c20, c21 = _threefry(s20, s21, zeros, ones)

    span = jnp.full((16,), _u32c(_BUFFER_SIZE))

    old_base = jnp.where(wid == 31, 0, 4 + jnp.minimum(wid * 128, 3760))

    def _idx_chunk(c, carry):
        j = (old_base + c * 16).astype(_U32) + lanes_u
        o0, o1 = _threefry(c20, c21, zeros, j)
        idx_v[pl.ds(c * 16, 16)] = ((o0 ^ o1) % span).astype(jnp.int32)
        return carry

    lax.fori_loop(0, 8, _idx_chunk, 0)

    @pl.when(wid < 31)
    def _gather_start():
        pltpu.async_copy(buf_hbm.at[idx_v], rows_v, sem)

    @pl.when(wid == 31)
    def _gather_mixed_start():
        pltpu.async_copy(buf_hbm.at[idx_v.at[pl.ds(0, 4)]],
                         new_v.at[pl.ds(4, 4)], sem)

    @pl.when((wid < 25) | (wid == 31))
    def _new_slab():
        start = jnp.where(wid == 31, 200, wid * 8)
        nchunks = jnp.where(wid == 31, 32, 64)

        def _chunk(c, carry):
            new_v[c // 8, pl.ds((c % 8) * 16, 16)] = _uniform_chunk(
                s10, s11, zeros, lanes_u, start * _D + c * 16)
            return carry

        lax.fori_loop(0, nchunks, _chunk, 0)

        @pl.when(wid == 31)
        def _mixed_wait():
            pltpu.make_async_copy(buf_hbm.at[idx_v.at[pl.ds(0, 4)]],
                                  new_v.at[pl.ds(4, 4)], sem).wait()

        pltpu.async_copy(new_v,
                         out_hbm.at[pl.ds(pl.multiple_of(start, 8), 8)],
                         sem_c)

    @pl.when(wid < 31)
    def _old_slab():
        start = 208 + jnp.minimum(wid * 128, 3760)
        pltpu.make_async_copy(buf_hbm.at[idx_v], rows_v, sem).wait()
        pltpu.sync_copy(rows_v,
                        out_hbm.at[pl.ds(pl.multiple_of(start, 8), 128)])

    @pl.when((wid < 25) | (wid == 31))
    def _new_slab_drain():
        start = jnp.where(wid == 31, 200, wid * 8)
        pltpu.make_async_copy(
            new_v, out_hbm.at[pl.ds(pl.multiple_of(start, 8), 8)],
            sem_c).wait()


_mesh = plsc.VectorSubcoreMesh(core_axis_name="c", subcore_axis_name="s")

_sample = functools.partial(
    pl.kernel,
    mesh=_mesh,
    out_type=jax.ShapeDtypeStruct((_NUM_CHAINS, _D), jnp.float32),
    scratch_types=[
        pltpu.VMEM((16,), jnp.int32),
        pltpu.VMEM((128,), jnp.int32),
        pltpu.VMEM((8, _D), jnp.float32),
        pltpu.VMEM((128, _D), jnp.float32),
        pltpu.SemaphoreType.DMA,
        pltpu.SemaphoreType.DMA,
    ],
)(_sc_body)


def kernel(buffer, key):
    kd = lax.bitcast_convert_type(jax.random.key_data(key), jnp.int32)
    return _sample(buffer, kd)
